# baseline (speedup 1.0000x reference)
import functools

import jax
import jax.numpy as jnp
import numpy as np
from jax import lax
from jax.experimental import pallas as pl
from jax.experimental.pallas import tpu as pltpu
from jax.experimental.pallas import tpu_sc as plsc

N = 10000
E = 160000
C = 128
A = 4
NB = 16
H = 4
NEU = 64

NP = 10112
NW = 32

def _sc_mesh():
    return plsc.VectorSubcoreMesh(core_axis_name="c", subcore_axis_name="s")


def _sc_gather(table, idx, chunk):
    B = idx.shape[0]
    V, D = table.shape
    bpw = B // NW
    nch = bpw // chunk
    assert nch % 2 == 0

    @functools.partial(
        pl.kernel, mesh=_sc_mesh(),
        out_type=jax.ShapeDtypeStruct((B, D), table.dtype),
        scratch_types=[
            pltpu.VMEM((chunk,), jnp.int32),
            pltpu.VMEM((chunk,), jnp.int32),
            pltpu.VMEM((2, chunk, D), table.dtype),
            pltpu.SemaphoreType.DMA,
            pltpu.SemaphoreType.DMA,
            pltpu.SemaphoreType.DMA,
            pltpu.SemaphoreType.DMA,
            pltpu.SemaphoreType.DMA,
        ],
    )
    def k(table_hbm, idx_hbm, out_hbm, idx_v0, idx_v1, rows_v,
          sem_i0, sem_i1, sem_g, sem_w0, sem_w1):
        wid = lax.axis_index("s") * 2 + lax.axis_index("c")
        base = wid * bpw
        idx_v = (idx_v0, idx_v1)
        sem_i = (sem_i0, sem_i1)
        sem_w = (sem_w0, sem_w1)

        for b in range(2):
            pltpu.async_copy(idx_hbm.at[pl.ds(base + b * chunk, chunk)],
                             idx_v[b], sem_i[b])

        @pl.loop(0, nch // 2)
        def _(go):
            for b in range(2):
                g = go * 2 + b
                off = base + g * chunk
                pltpu.make_async_copy(idx_hbm.at[pl.ds(off, chunk)],
                                      idx_v[b], sem_i[b]).wait()
                @pl.when(g >= 2)
                def _():
                    pltpu.make_async_copy(rows_v.at[b],
                                          out_hbm.at[pl.ds(base, chunk)],
                                          sem_w[b]).wait()
                pltpu.async_copy(table_hbm.at[idx_v[b]], rows_v.at[b],
                                 sem_g).wait()
                @pl.when(go < nch // 2 - 1)
                def _():
                    pltpu.async_copy(
                        idx_hbm.at[pl.ds(off + 2 * chunk, chunk)],
                        idx_v[b], sem_i[b])
                pltpu.async_copy(rows_v.at[b], out_hbm.at[pl.ds(off, chunk)],
                                 sem_w[b])

        for b in range(2):
            pltpu.make_async_copy(rows_v.at[b],
                                  out_hbm.at[pl.ds(base, chunk)],
                                  sem_w[b]).wait()

    return k(table, idx)


def _sc_scatter_add2(vout, expe, idx, zeros_c, zeros_h, chunk):
    B = idx.shape[0]
    bpw = B // NW
    nch = bpw // chunk
    assert nch % 2 == 1
    rows_pt = NP // 16

    @functools.partial(
        pl.kernel, mesh=_sc_mesh(),
        compiler_params=pltpu.CompilerParams(use_tc_tiling_on_sc=False),
        out_type=(jax.ShapeDtypeStruct((2 * NP, C), jnp.float32),
                  jax.ShapeDtypeStruct((2 * NP, 16), jnp.float32)),
        scratch_types=[
            pltpu.VMEM((chunk,), jnp.int32),
            pltpu.VMEM((chunk,), jnp.int32),
            pltpu.VMEM((2, chunk, C), jnp.float32),
            pltpu.VMEM((2, chunk, 16), jnp.float32),
            pltpu.VMEM_SHARED((NP, C), jnp.float32),
            pltpu.VMEM_SHARED((NP, 16), jnp.float32),
            pltpu.SemaphoreType.DMA,
            pltpu.SemaphoreType.DMA,
        ],
    )
    def k(vout_hbm, expe_hbm, idx_hbm, zc_hbm, zh_hbm, outc_hbm, outh_hbm,
          idx_v0, idx_v1, vc_v, vh_v, acc_c, acc_h, sem_l0, sem_l1):
        idx_v = (idx_v0, idx_v1)
        cid = lax.axis_index("c")
        sid = lax.axis_index("s")
        wid = sid * 2 + cid
        r0 = sid * rows_pt
        base = wid * bpw
        sem_l = (sem_l0, sem_l1)

        def issue_loads(g, b):
            off = base + g * chunk
            pltpu.async_copy(idx_hbm.at[pl.ds(off, chunk)], idx_v[b],
                             sem_l[b])
            pltpu.async_copy(vout_hbm.at[pl.ds(off, chunk)], vc_v.at[b],
                             sem_l[b])
            pltpu.async_copy(expe_hbm.at[pl.ds(off, chunk)], vh_v.at[b],
                             sem_l[b])

        def wait_loads(g, b):
            off = base + g * chunk
            pltpu.make_async_copy(idx_hbm.at[pl.ds(off, chunk)], idx_v[b],
                                  sem_l[b]).wait()
            pltpu.make_async_copy(vout_hbm.at[pl.ds(off, chunk)], vc_v.at[b],
                                  sem_l[b]).wait()
            pltpu.make_async_copy(expe_hbm.at[pl.ds(off, chunk)], vh_v.at[b],
                                  sem_l[b]).wait()

        def do_adds(b):
            pltpu.sync_copy(vc_v.at[b], acc_c.at[idx_v[b]], add=True)
            pltpu.sync_copy(vh_v.at[b], acc_h.at[idx_v[b]], add=True)

        pltpu.sync_copy(zc_hbm.at[pl.ds(r0, rows_pt)],
                        acc_c.at[pl.ds(r0, rows_pt)])
        pltpu.sync_copy(zh_hbm.at[pl.ds(r0, rows_pt)],
                        acc_h.at[pl.ds(r0, rows_pt)])
        plsc.subcore_barrier()

        issue_loads(0, 0)

        @pl.loop(0, (nch - 1) // 2)
        def _(go):
            for b in range(2):
                g = go * 2 + b
                issue_loads_g1 = g + 1
                pltpu.async_copy(
                    idx_hbm.at[pl.ds(base + issue_loads_g1 * chunk, chunk)],
                    idx_v[1 - b], sem_l[1 - b])
                pltpu.async_copy(
                    vout_hbm.at[pl.ds(base + issue_loads_g1 * chunk, chunk)],
                    vc_v.at[1 - b], sem_l[1 - b])
                pltpu.async_copy(
                    expe_hbm.at[pl.ds(base + issue_loads_g1 * chunk, chunk)],
                    vh_v.at[1 - b], sem_l[1 - b])
                wait_loads(g, b)
                do_adds(b)

        wait_loads(nch - 1, (nch - 1) % 2)
        do_adds((nch - 1) % 2)

        plsc.subcore_barrier()
        ro = cid * NP + r0
        pltpu.sync_copy(acc_c.at[pl.ds(r0, rows_pt)],
                        outc_hbm.at[pl.ds(ro, rows_pt)])
        pltpu.sync_copy(acc_h.at[pl.ds(r0, rows_pt)],
                        outh_hbm.at[pl.ds(ro, rows_pt)])

    outc, outh = k(vout, expe, idx, zeros_c, zeros_h)
    return outc.reshape(2, NP, C), outh.reshape(2, NP, 16)


BE = 2000


def _edge_logits(esa, ea, cut2, src, dst, w1k, w2k, wkt, wlt):
    def body(esa_r, ea_r, cut_r, src_r, dst_r, w1k_r, w2k_r, wkt_r, wlt_r, out_r):
        hk = jnp.dot(esa_r[...], w1k_r[...], preferred_element_type=jnp.float32)
        hk = jnp.dot(jax.nn.relu(hk), w2k_r[...], preferred_element_type=jnp.float32)
        hk = jax.nn.relu(hk).astype(jnp.bfloat16)
        eav = ea_r[...]
        m = None
        for v in range(A):
            t = jnp.dot(hk, wkt_r[v], preferred_element_type=jnp.float32)
            t = t * eav[:, v:v + 1]
            m = t if m is None else m + t
        ek = (src_r[...] * m).astype(jnp.bfloat16)
        dstv = dst_r[...]
        cols = []
        for h in range(H):
            t = jnp.dot(ek, wlt_r[h], preferred_element_type=jnp.float32)
            cols.append(jnp.sum(dstv * t, axis=1, keepdims=True))
        logit = jnp.concatenate(cols, axis=1)
        e4 = cut_r[...] * jnp.exp(logit)
        pad = jnp.zeros((e4.shape[0], 16 - H), e4.dtype)
        out_r[...] = jnp.concatenate([e4, pad], axis=1)

    return pl.pallas_call(
        body,
        grid=(E // BE,),
        in_specs=[
            pl.BlockSpec((BE, NB), lambda i: (i, 0)),
            pl.BlockSpec((BE, A), lambda i: (i, 0)),
            pl.BlockSpec((BE, 1), lambda i: (i, 0)),
            pl.BlockSpec((BE, C), lambda i: (i, 0)),
            pl.BlockSpec((BE, C), lambda i: (i, 0)),
            pl.BlockSpec((NB, NEU), lambda i: (0, 0)),
            pl.BlockSpec((NEU, NEU), lambda i: (0, 0)),
            pl.BlockSpec((A, NEU, C), lambda i: (0, 0, 0)),
            pl.BlockSpec((H, C, C), lambda i: (0, 0, 0)),
        ],
        out_specs=pl.BlockSpec((BE, 16), lambda i: (i, 0)),
        out_shape=jax.ShapeDtypeStruct((E, 16), jnp.float32),
    )(esa, ea, cut2, src, dst, w1k, w2k, wkt, wlt)


def _edge_values(esa, ea, src, expe, w1v, w2v, wvt):
    def body(esa_r, ea_r, src_r, exp_r, w1v_r, w2v_r, wvt_r, out_r):
        hv = jnp.dot(esa_r[...], w1v_r[...], preferred_element_type=jnp.float32)
        hv = jnp.dot(jax.nn.relu(hv), w2v_r[...], preferred_element_type=jnp.float32)
        hv = jax.nn.relu(hv).astype(jnp.bfloat16)
        eav = ea_r[...]
        m = None
        for v in range(A):
            t = jnp.dot(hv, wvt_r[v], preferred_element_type=jnp.float32)
            t = t * eav[:, v:v + 1]
            m = t if m is None else m + t
        ev = src_r[...] * m
        w16 = jnp.sqrt(exp_r[...])
        CH = C // H
        pieces = [ev[:, h * CH:(h + 1) * CH] * w16[:, h:h + 1] for h in range(H)]
        out_r[...] = jnp.concatenate(pieces, axis=1)

    return pl.pallas_call(
        body,
        grid=(E // BE,),
        in_specs=[
            pl.BlockSpec((BE, NB), lambda i: (i, 0)),
            pl.BlockSpec((BE, A), lambda i: (i, 0)),
            pl.BlockSpec((BE, C), lambda i: (i, 0)),
            pl.BlockSpec((BE, 16), lambda i: (i, 0)),
            pl.BlockSpec((NB, NEU), lambda i: (0, 0)),
            pl.BlockSpec((NEU, NEU), lambda i: (0, 0)),
            pl.BlockSpec((A, NEU, C), lambda i: (0, 0, 0)),
        ],
        out_specs=pl.BlockSpec((BE, C), lambda i: (i, 0)),
        out_shape=jax.ShapeDtypeStruct((E, C), jnp.float32),
    )(esa, ea, src, expe, w1v, w2v, wvt)


def _final_linear(n0, n1, z0, z1, wl):
    def body(n0_r, n1_r, z0_r, z1_r, wl_r, out_r):
        z = z0_r[...] + z1_r[...]
        w = jnp.where(z == 0.0, 1.0, lax.rsqrt(z))
        ns = n0_r[...] + n1_r[...]
        CH = C // H
        pieces = [ns[:, h * CH:(h + 1) * CH] * w[:, h:h + 1] for h in range(H)]
        scaled = jnp.concatenate(pieces, axis=1)
        out_r[...] = jnp.dot(scaled, wl_r[...],
                             preferred_element_type=jnp.float32)

    return pl.pallas_call(
        body,
        out_shape=jax.ShapeDtypeStruct((N, C), jnp.float32),
    )(n0, n1, z0, z1, wl)


def kernel(edge_src, edge_dst, edge_scalar_attr, edge_weight_cutoff, edge_attr,
           node_feat, W1k, W2k, W1v, W2v, wk, wv, wlogit, wlin):
    edge_src = edge_src.astype(jnp.int32)
    edge_dst = edge_dst.astype(jnp.int32)

    w1k = W1k / np.sqrt(NB)
    w2k = W2k / np.sqrt(NEU)
    w1v = W1v / np.sqrt(NB)
    w2v = W2v / np.sqrt(NEU)
    wkt = (jnp.transpose(wk, (2, 0, 1)) / (np.sqrt(NEU) * np.sqrt(A))).astype(jnp.bfloat16)
    wvt = (jnp.transpose(wv, (2, 0, 1)) / (np.sqrt(NEU) * np.sqrt(A))).astype(jnp.bfloat16)
    wlt = (jnp.transpose(wlogit, (2, 1, 0)) / C).astype(jnp.bfloat16)
    wl = wlin / np.sqrt(C)

    cut2 = edge_weight_cutoff.reshape(E, 1)

    both = _sc_gather(node_feat, jnp.concatenate([edge_src, edge_dst]),
                      chunk=200)
    src_feat = both[:E]
    dst_feat = both[E:]

    expe = _edge_logits(edge_scalar_attr, edge_attr, cut2, src_feat, dst_feat,
                        w1k, w2k, wkt, wlt)

    vout = _edge_values(edge_scalar_attr, edge_attr, src_feat, expe,
                        w1v, w2v, wvt)

    zC = jnp.zeros((NP, C), jnp.float32)
    z16 = jnp.zeros((NP, 16), jnp.float32)
    npart, zpart = _sc_scatter_add2(vout, expe, edge_dst, zC, z16, chunk=40)

    return _final_linear(npart[0, :N], npart[1, :N],
                         zpart[0, :N], zpart[1, :N], wl)


# speedup vs baseline: 1.1065x; 1.1065x over previous
import functools

import jax
import jax.numpy as jnp
import numpy as np
from jax import lax
from jax.experimental import pallas as pl
from jax.experimental.pallas import tpu as pltpu
from jax.experimental.pallas import tpu_sc as plsc

N = 10000
E = 160000
C = 128
A = 4
NB = 16
H = 4
NEU = 64

NP = 10112
NW = 32

def _sc_mesh():
    return plsc.VectorSubcoreMesh(core_axis_name="c", subcore_axis_name="s")


def _sc_gather(table, idx, chunk):
    B = idx.shape[0]
    V, D = table.shape
    bpw = B // NW
    nch = bpw // chunk
    assert nch % 2 == 0

    @functools.partial(
        pl.kernel, mesh=_sc_mesh(),
        out_type=jax.ShapeDtypeStruct((B, D), table.dtype),
        scratch_types=[
            pltpu.VMEM((chunk,), jnp.int32),
            pltpu.VMEM((chunk,), jnp.int32),
            pltpu.VMEM((2, chunk, D), table.dtype),
            pltpu.SemaphoreType.DMA,
            pltpu.SemaphoreType.DMA,
            pltpu.SemaphoreType.DMA,
            pltpu.SemaphoreType.DMA,
            pltpu.SemaphoreType.DMA,
        ],
    )
    def k(table_hbm, idx_hbm, out_hbm, idx_v0, idx_v1, rows_v,
          sem_i0, sem_i1, sem_g, sem_w0, sem_w1):
        wid = lax.axis_index("s") * 2 + lax.axis_index("c")
        base = wid * bpw
        idx_v = (idx_v0, idx_v1)
        sem_i = (sem_i0, sem_i1)
        sem_w = (sem_w0, sem_w1)

        for b in range(2):
            pltpu.async_copy(idx_hbm.at[pl.ds(base + b * chunk, chunk)],
                             idx_v[b], sem_i[b])

        @pl.loop(0, nch // 2)
        def _(go):
            for b in range(2):
                g = go * 2 + b
                off = base + g * chunk
                pltpu.make_async_copy(idx_hbm.at[pl.ds(off, chunk)],
                                      idx_v[b], sem_i[b]).wait()
                @pl.when(g >= 2)
                def _():
                    pltpu.make_async_copy(rows_v.at[b],
                                          out_hbm.at[pl.ds(base, chunk)],
                                          sem_w[b]).wait()
                pltpu.async_copy(table_hbm.at[idx_v[b]], rows_v.at[b],
                                 sem_g).wait()
                @pl.when(go < nch // 2 - 1)
                def _():
                    pltpu.async_copy(
                        idx_hbm.at[pl.ds(off + 2 * chunk, chunk)],
                        idx_v[b], sem_i[b])
                pltpu.async_copy(rows_v.at[b], out_hbm.at[pl.ds(off, chunk)],
                                 sem_w[b])

        for b in range(2):
            pltpu.make_async_copy(rows_v.at[b],
                                  out_hbm.at[pl.ds(base, chunk)],
                                  sem_w[b]).wait()

    return k(table, idx)


def _sc_scatter_add2(vout, expe, idx, zeros_c, zeros_h, chunk):
    B = idx.shape[0]
    bpw = B // NW
    nch = bpw // chunk
    assert nch % 2 == 1
    rows_pt = NP // 16

    @functools.partial(
        pl.kernel, mesh=_sc_mesh(),
        compiler_params=pltpu.CompilerParams(use_tc_tiling_on_sc=False),
        out_type=(jax.ShapeDtypeStruct((2 * NP, C), jnp.float32),
                  jax.ShapeDtypeStruct((2 * NP, 16), jnp.float32)),
        scratch_types=[
            pltpu.VMEM((chunk,), jnp.int32),
            pltpu.VMEM((chunk,), jnp.int32),
            pltpu.VMEM((2, chunk, C), jnp.float32),
            pltpu.VMEM((2, chunk, 16), jnp.float32),
            pltpu.VMEM_SHARED((NP, C), jnp.float32),
            pltpu.VMEM_SHARED((NP, 16), jnp.float32),
            pltpu.SemaphoreType.DMA,
            pltpu.SemaphoreType.DMA,
        ],
    )
    def k(vout_hbm, expe_hbm, idx_hbm, zc_hbm, zh_hbm, outc_hbm, outh_hbm,
          idx_v0, idx_v1, vc_v, vh_v, acc_c, acc_h, sem_l0, sem_l1):
        idx_v = (idx_v0, idx_v1)
        cid = lax.axis_index("c")
        sid = lax.axis_index("s")
        wid = sid * 2 + cid
        r0 = sid * rows_pt
        base = wid * bpw
        sem_l = (sem_l0, sem_l1)

        def issue_loads(g, b):
            off = base + g * chunk
            pltpu.async_copy(idx_hbm.at[pl.ds(off, chunk)], idx_v[b],
                             sem_l[b])
            pltpu.async_copy(vout_hbm.at[pl.ds(off, chunk)], vc_v.at[b],
                             sem_l[b])
            pltpu.async_copy(expe_hbm.at[pl.ds(off, chunk)], vh_v.at[b],
                             sem_l[b])

        def wait_loads(g, b):
            off = base + g * chunk
            pltpu.make_async_copy(idx_hbm.at[pl.ds(off, chunk)], idx_v[b],
                                  sem_l[b]).wait()
            pltpu.make_async_copy(vout_hbm.at[pl.ds(off, chunk)], vc_v.at[b],
                                  sem_l[b]).wait()
            pltpu.make_async_copy(expe_hbm.at[pl.ds(off, chunk)], vh_v.at[b],
                                  sem_l[b]).wait()

        def do_adds(b):
            pltpu.sync_copy(vc_v.at[b], acc_c.at[idx_v[b]], add=True)
            pltpu.sync_copy(vh_v.at[b], acc_h.at[idx_v[b]], add=True)

        pltpu.sync_copy(zc_hbm.at[pl.ds(r0, rows_pt)],
                        acc_c.at[pl.ds(r0, rows_pt)])
        pltpu.sync_copy(zh_hbm.at[pl.ds(r0, rows_pt)],
                        acc_h.at[pl.ds(r0, rows_pt)])
        plsc.subcore_barrier()

        issue_loads(0, 0)

        @pl.loop(0, (nch - 1) // 2)
        def _(go):
            for b in range(2):
                g = go * 2 + b
                issue_loads_g1 = g + 1
                pltpu.async_copy(
                    idx_hbm.at[pl.ds(base + issue_loads_g1 * chunk, chunk)],
                    idx_v[1 - b], sem_l[1 - b])
                pltpu.async_copy(
                    vout_hbm.at[pl.ds(base + issue_loads_g1 * chunk, chunk)],
                    vc_v.at[1 - b], sem_l[1 - b])
                pltpu.async_copy(
                    expe_hbm.at[pl.ds(base + issue_loads_g1 * chunk, chunk)],
                    vh_v.at[1 - b], sem_l[1 - b])
                wait_loads(g, b)
                do_adds(b)

        wait_loads(nch - 1, (nch - 1) % 2)
        do_adds((nch - 1) % 2)

        plsc.subcore_barrier()
        ro = cid * NP + r0
        pltpu.sync_copy(acc_c.at[pl.ds(r0, rows_pt)],
                        outc_hbm.at[pl.ds(ro, rows_pt)])
        pltpu.sync_copy(acc_h.at[pl.ds(r0, rows_pt)],
                        outh_hbm.at[pl.ds(ro, rows_pt)])

    outc, outh = k(vout, expe, idx, zeros_c, zeros_h)
    return outc.reshape(2, NP, C), outh.reshape(2, NP, 16)


BE = 2000


def _edge_logits(esa, ea, cut2, src, dst, w1k, w2k, wkt, wlt):
    def body(esa_r, ea_r, cut_r, src_r, dst_r, w1k_r, w2k_r, wkt_r, wlt_r, out_r):
        hk = jnp.dot(esa_r[...], w1k_r[...], preferred_element_type=jnp.float32)
        hk = jnp.dot(jax.nn.relu(hk), w2k_r[...], preferred_element_type=jnp.float32)
        hk = jax.nn.relu(hk).astype(jnp.bfloat16)
        eav = ea_r[...]
        m = None
        for v in range(A):
            t = jnp.dot(hk, wkt_r[v], preferred_element_type=jnp.float32)
            t = t * eav[:, v:v + 1]
            m = t if m is None else m + t
        ek = (src_r[...] * m).astype(jnp.bfloat16)
        dstv = dst_r[...]
        cols = []
        for h in range(H):
            t = jnp.dot(ek, wlt_r[h], preferred_element_type=jnp.float32)
            cols.append(jnp.sum(dstv * t, axis=1, keepdims=True))
        logit = jnp.concatenate(cols, axis=1)
        e4 = cut_r[...] * jnp.exp(logit)
        pad = jnp.zeros((e4.shape[0], 16 - H), e4.dtype)
        out_r[...] = jnp.concatenate([e4, pad], axis=1)

    return pl.pallas_call(
        body,
        grid=(E // BE,),
        in_specs=[
            pl.BlockSpec((BE, NB), lambda i: (i, 0)),
            pl.BlockSpec((BE, A), lambda i: (i, 0)),
            pl.BlockSpec((BE, 1), lambda i: (i, 0)),
            pl.BlockSpec((BE, C), lambda i: (i, 0)),
            pl.BlockSpec((BE, C), lambda i: (i + E // BE, 0)),
            pl.BlockSpec((NB, NEU), lambda i: (0, 0)),
            pl.BlockSpec((NEU, NEU), lambda i: (0, 0)),
            pl.BlockSpec((A, NEU, C), lambda i: (0, 0, 0)),
            pl.BlockSpec((H, C, C), lambda i: (0, 0, 0)),
        ],
        out_specs=pl.BlockSpec((BE, 16), lambda i: (i, 0)),
        out_shape=jax.ShapeDtypeStruct((E, 16), jnp.float32),
    )(esa, ea, cut2, src, dst, w1k, w2k, wkt, wlt)


def _edge_values(esa, ea, src, expe, w1v, w2v, wvt):
    def body(esa_r, ea_r, src_r, exp_r, w1v_r, w2v_r, wvt_r, out_r):
        hv = jnp.dot(esa_r[...], w1v_r[...], preferred_element_type=jnp.float32)
        hv = jnp.dot(jax.nn.relu(hv), w2v_r[...], preferred_element_type=jnp.float32)
        hv = jax.nn.relu(hv).astype(jnp.bfloat16)
        eav = ea_r[...]
        m = None
        for v in range(A):
            t = jnp.dot(hv, wvt_r[v], preferred_element_type=jnp.float32)
            t = t * eav[:, v:v + 1]
            m = t if m is None else m + t
        ev = src_r[...] * m
        w16 = jnp.sqrt(exp_r[...])
        CH = C // H
        pieces = [ev[:, h * CH:(h + 1) * CH] * w16[:, h:h + 1] for h in range(H)]
        out_r[...] = jnp.concatenate(pieces, axis=1)

    return pl.pallas_call(
        body,
        grid=(E // BE,),
        in_specs=[
            pl.BlockSpec((BE, NB), lambda i: (i, 0)),
            pl.BlockSpec((BE, A), lambda i: (i, 0)),
            pl.BlockSpec((BE, C), lambda i: (i, 0)),
            pl.BlockSpec((BE, 16), lambda i: (i, 0)),
            pl.BlockSpec((NB, NEU), lambda i: (0, 0)),
            pl.BlockSpec((NEU, NEU), lambda i: (0, 0)),
            pl.BlockSpec((A, NEU, C), lambda i: (0, 0, 0)),
        ],
        out_specs=pl.BlockSpec((BE, C), lambda i: (i, 0)),
        out_shape=jax.ShapeDtypeStruct((E, C), jnp.float32),
    )(esa, ea, src, expe, w1v, w2v, wvt)


def _final_linear(n0, n1, z0, z1, wl):
    def body(n0_r, n1_r, z0_r, z1_r, wl_r, out_r):
        z = z0_r[...] + z1_r[...]
        w = jnp.where(z == 0.0, 1.0, lax.rsqrt(z))
        ns = n0_r[...] + n1_r[...]
        CH = C // H
        pieces = [ns[:, h * CH:(h + 1) * CH] * w[:, h:h + 1] for h in range(H)]
        scaled = jnp.concatenate(pieces, axis=1)
        out_r[...] = jnp.dot(scaled, wl_r[...],
                             preferred_element_type=jnp.float32)

    return pl.pallas_call(
        body,
        out_shape=jax.ShapeDtypeStruct((N, C), jnp.float32),
    )(n0, n1, z0, z1, wl)


def kernel(edge_src, edge_dst, edge_scalar_attr, edge_weight_cutoff, edge_attr,
           node_feat, W1k, W2k, W1v, W2v, wk, wv, wlogit, wlin):
    edge_src = edge_src.astype(jnp.int32)
    edge_dst = edge_dst.astype(jnp.int32)

    w1k = W1k / np.sqrt(NB)
    w2k = W2k / np.sqrt(NEU)
    w1v = W1v / np.sqrt(NB)
    w2v = W2v / np.sqrt(NEU)
    wkt = (jnp.transpose(wk, (2, 0, 1)) / (np.sqrt(NEU) * np.sqrt(A))).astype(jnp.bfloat16)
    wvt = (jnp.transpose(wv, (2, 0, 1)) / (np.sqrt(NEU) * np.sqrt(A))).astype(jnp.bfloat16)
    wlt = (jnp.transpose(wlogit, (2, 1, 0)) / C).astype(jnp.bfloat16)
    wl = wlin / np.sqrt(C)

    cut2 = edge_weight_cutoff.reshape(E, 1)

    both = _sc_gather(node_feat, jnp.concatenate([edge_src, edge_dst]),
                      chunk=200)

    expe = _edge_logits(edge_scalar_attr, edge_attr, cut2, both, both,
                        w1k, w2k, wkt, wlt)

    vout = _edge_values(edge_scalar_attr, edge_attr, both, expe,
                        w1v, w2v, wvt)

    zC = jnp.zeros((NP, C), jnp.float32)
    z16 = jnp.zeros((NP, 16), jnp.float32)
    npart, zpart = _sc_scatter_add2(vout, expe, edge_dst, zC, z16, chunk=40)

    return _final_linear(npart[0, :N], npart[1, :N],
                         zpart[0, :N], zpart[1, :N], wl)


# speedup vs baseline: 1.1197x; 1.0119x over previous
import functools

import jax
import jax.numpy as jnp
import numpy as np
from jax import lax
from jax.experimental import pallas as pl
from jax.experimental.pallas import tpu as pltpu
from jax.experimental.pallas import tpu_sc as plsc

N = 10000
E = 160000
C = 128
A = 4
NB = 16
H = 4
NEU = 64

NP = 10112
NW = 32

def _sc_mesh():
    return plsc.VectorSubcoreMesh(core_axis_name="c", subcore_axis_name="s")


def _sc_gather(table, idx, chunk):
    B = idx.shape[0]
    V, D = table.shape
    bpw = B // NW
    nch = bpw // chunk
    assert nch % 2 == 0

    @functools.partial(
        pl.kernel, mesh=_sc_mesh(),
        out_type=jax.ShapeDtypeStruct((B, D), table.dtype),
        scratch_types=[
            pltpu.VMEM((chunk,), jnp.int32),
            pltpu.VMEM((chunk,), jnp.int32),
            pltpu.VMEM((2, chunk, D), table.dtype),
            pltpu.SemaphoreType.DMA,
            pltpu.SemaphoreType.DMA,
            pltpu.SemaphoreType.DMA,
            pltpu.SemaphoreType.DMA,
            pltpu.SemaphoreType.DMA,
        ],
    )
    def k(table_hbm, idx_hbm, out_hbm, idx_v0, idx_v1, rows_v,
          sem_i0, sem_i1, sem_g, sem_w0, sem_w1):
        wid = lax.axis_index("s") * 2 + lax.axis_index("c")
        base = wid * bpw
        idx_v = (idx_v0, idx_v1)
        sem_i = (sem_i0, sem_i1)
        sem_w = (sem_w0, sem_w1)

        for b in range(2):
            pltpu.async_copy(idx_hbm.at[pl.ds(base + b * chunk, chunk)],
                             idx_v[b], sem_i[b])

        @pl.loop(0, nch // 2)
        def _(go):
            for b in range(2):
                g = go * 2 + b
                off = base + g * chunk
                pltpu.make_async_copy(idx_hbm.at[pl.ds(off, chunk)],
                                      idx_v[b], sem_i[b]).wait()
                @pl.when(g >= 2)
                def _():
                    pltpu.make_async_copy(rows_v.at[b],
                                          out_hbm.at[pl.ds(base, chunk)],
                                          sem_w[b]).wait()
                pltpu.async_copy(table_hbm.at[idx_v[b]], rows_v.at[b],
                                 sem_g).wait()
                @pl.when(go < nch // 2 - 1)
                def _():
                    pltpu.async_copy(
                        idx_hbm.at[pl.ds(off + 2 * chunk, chunk)],
                        idx_v[b], sem_i[b])
                pltpu.async_copy(rows_v.at[b], out_hbm.at[pl.ds(off, chunk)],
                                 sem_w[b])

        for b in range(2):
            pltpu.make_async_copy(rows_v.at[b],
                                  out_hbm.at[pl.ds(base, chunk)],
                                  sem_w[b]).wait()

    return k(table, idx)


def _sc_scatter_add2(vout, expe, idx, zeros_c, zeros_h, chunk):
    B = idx.shape[0]
    bpw = B // NW
    nch = bpw // chunk
    assert nch % 2 == 1
    rows_pt = NP // 16

    @functools.partial(
        pl.kernel, mesh=_sc_mesh(),
        compiler_params=pltpu.CompilerParams(use_tc_tiling_on_sc=False),
        out_type=(jax.ShapeDtypeStruct((2 * NP, C), jnp.float32),
                  jax.ShapeDtypeStruct((2 * NP, 16), jnp.float32)),
        scratch_types=[
            pltpu.VMEM((chunk,), jnp.int32),
            pltpu.VMEM((chunk,), jnp.int32),
            pltpu.VMEM((2, chunk, C), jnp.float32),
            pltpu.VMEM((2, chunk, 16), jnp.float32),
            pltpu.VMEM_SHARED((NP, C), jnp.float32),
            pltpu.VMEM_SHARED((NP, 16), jnp.float32),
            pltpu.SemaphoreType.DMA,
            pltpu.SemaphoreType.DMA,
        ],
    )
    def k(vout_hbm, expe_hbm, idx_hbm, zc_hbm, zh_hbm, outc_hbm, outh_hbm,
          idx_v0, idx_v1, vc_v, vh_v, acc_c, acc_h, sem_l0, sem_l1):
        idx_v = (idx_v0, idx_v1)
        cid = lax.axis_index("c")
        sid = lax.axis_index("s")
        wid = sid * 2 + cid
        r0 = sid * rows_pt
        base = wid * bpw
        sem_l = (sem_l0, sem_l1)

        def issue_loads(g, b):
            off = base + g * chunk
            pltpu.async_copy(idx_hbm.at[pl.ds(off, chunk)], idx_v[b],
                             sem_l[b])
            pltpu.async_copy(vout_hbm.at[pl.ds(off, chunk)], vc_v.at[b],
                             sem_l[b])
            pltpu.async_copy(expe_hbm.at[pl.ds(off, chunk)], vh_v.at[b],
                             sem_l[b])

        def wait_loads(g, b):
            off = base + g * chunk
            pltpu.make_async_copy(idx_hbm.at[pl.ds(off, chunk)], idx_v[b],
                                  sem_l[b]).wait()
            pltpu.make_async_copy(vout_hbm.at[pl.ds(off, chunk)], vc_v.at[b],
                                  sem_l[b]).wait()
            pltpu.make_async_copy(expe_hbm.at[pl.ds(off, chunk)], vh_v.at[b],
                                  sem_l[b]).wait()

        def do_adds(b):
            pltpu.sync_copy(vc_v.at[b], acc_c.at[idx_v[b]], add=True)
            pltpu.sync_copy(vh_v.at[b], acc_h.at[idx_v[b]], add=True)

        pltpu.sync_copy(zc_hbm.at[pl.ds(r0, rows_pt)],
                        acc_c.at[pl.ds(r0, rows_pt)])
        pltpu.sync_copy(zh_hbm.at[pl.ds(r0, rows_pt)],
                        acc_h.at[pl.ds(r0, rows_pt)])
        plsc.subcore_barrier()

        issue_loads(0, 0)

        @pl.loop(0, (nch - 1) // 2)
        def _(go):
            for b in range(2):
                g = go * 2 + b
                issue_loads_g1 = g + 1
                pltpu.async_copy(
                    idx_hbm.at[pl.ds(base + issue_loads_g1 * chunk, chunk)],
                    idx_v[1 - b], sem_l[1 - b])
                pltpu.async_copy(
                    vout_hbm.at[pl.ds(base + issue_loads_g1 * chunk, chunk)],
                    vc_v.at[1 - b], sem_l[1 - b])
                pltpu.async_copy(
                    expe_hbm.at[pl.ds(base + issue_loads_g1 * chunk, chunk)],
                    vh_v.at[1 - b], sem_l[1 - b])
                wait_loads(g, b)
                do_adds(b)

        wait_loads(nch - 1, (nch - 1) % 2)
        do_adds((nch - 1) % 2)

        plsc.subcore_barrier()
        ro = cid * NP + r0
        pltpu.sync_copy(acc_c.at[pl.ds(r0, rows_pt)],
                        outc_hbm.at[pl.ds(ro, rows_pt)])
        pltpu.sync_copy(acc_h.at[pl.ds(r0, rows_pt)],
                        outh_hbm.at[pl.ds(ro, rows_pt)])

    outc, outh = k(vout, expe, idx, zeros_c, zeros_h)
    return outc.reshape(2, NP, C), outh.reshape(2, NP, 16)


BE = 2000


def _edge_logits(esa, ea, cut2, src, dst, w1k, w2k, wkt, wlt):
    def body(esa_r, ea_r, cut_r, src_r, dst_r, w1k_r, w2k_r, wkt_r, wlt_r, out_r):
        hk = jnp.dot(esa_r[...], w1k_r[...], preferred_element_type=jnp.float32)
        hk = jnp.dot(jax.nn.relu(hk), w2k_r[...], preferred_element_type=jnp.float32)
        hk = jax.nn.relu(hk).astype(jnp.bfloat16)
        eav = ea_r[...].astype(jnp.bfloat16)
        hk2 = jnp.concatenate([hk * eav[:, v:v + 1] for v in range(A)],
                              axis=1)
        m = jnp.dot(hk2, wkt_r[...], preferred_element_type=jnp.float32)
        ek = (src_r[...] * m).astype(jnp.bfloat16)
        dstv = dst_r[...]
        tt = jnp.dot(ek, wlt_r[...], preferred_element_type=jnp.float32)
        cols = [jnp.sum(dstv * tt[:, h * C:(h + 1) * C], axis=1,
                        keepdims=True) for h in range(H)]
        logit = jnp.concatenate(cols, axis=1)
        e4 = cut_r[...] * jnp.exp(logit)
        pad = jnp.zeros((e4.shape[0], 16 - H), e4.dtype)
        out_r[...] = jnp.concatenate([e4, pad], axis=1)

    return pl.pallas_call(
        body,
        grid=(E // BE,),
        in_specs=[
            pl.BlockSpec((BE, NB), lambda i: (i, 0)),
            pl.BlockSpec((BE, A), lambda i: (i, 0)),
            pl.BlockSpec((BE, 1), lambda i: (i, 0)),
            pl.BlockSpec((BE, C), lambda i: (i, 0)),
            pl.BlockSpec((BE, C), lambda i: (i + E // BE, 0)),
            pl.BlockSpec((NB, NEU), lambda i: (0, 0)),
            pl.BlockSpec((NEU, NEU), lambda i: (0, 0)),
            pl.BlockSpec((A * NEU, C), lambda i: (0, 0)),
            pl.BlockSpec((C, H * C), lambda i: (0, 0)),
        ],
        out_specs=pl.BlockSpec((BE, 16), lambda i: (i, 0)),
        out_shape=jax.ShapeDtypeStruct((E, 16), jnp.float32),
    )(esa, ea, cut2, src, dst, w1k, w2k, wkt, wlt)


def _edge_values(esa, ea, src, expe, w1v, w2v, wvt):
    def body(esa_r, ea_r, src_r, exp_r, w1v_r, w2v_r, wvt_r, out_r):
        hv = jnp.dot(esa_r[...], w1v_r[...], preferred_element_type=jnp.float32)
        hv = jnp.dot(jax.nn.relu(hv), w2v_r[...], preferred_element_type=jnp.float32)
        hv = jax.nn.relu(hv).astype(jnp.bfloat16)
        eav = ea_r[...].astype(jnp.bfloat16)
        hv2 = jnp.concatenate([hv * eav[:, v:v + 1] for v in range(A)],
                              axis=1)
        m = jnp.dot(hv2, wvt_r[...], preferred_element_type=jnp.float32)
        ev = src_r[...] * m
        w16 = jnp.sqrt(exp_r[...])
        CH = C // H
        pieces = [ev[:, h * CH:(h + 1) * CH] * w16[:, h:h + 1] for h in range(H)]
        out_r[...] = jnp.concatenate(pieces, axis=1)

    return pl.pallas_call(
        body,
        grid=(E // BE,),
        in_specs=[
            pl.BlockSpec((BE, NB), lambda i: (i, 0)),
            pl.BlockSpec((BE, A), lambda i: (i, 0)),
            pl.BlockSpec((BE, C), lambda i: (i, 0)),
            pl.BlockSpec((BE, 16), lambda i: (i, 0)),
            pl.BlockSpec((NB, NEU), lambda i: (0, 0)),
            pl.BlockSpec((NEU, NEU), lambda i: (0, 0)),
            pl.BlockSpec((A * NEU, C), lambda i: (0, 0)),
        ],
        out_specs=pl.BlockSpec((BE, C), lambda i: (i, 0)),
        out_shape=jax.ShapeDtypeStruct((E, C), jnp.float32),
    )(esa, ea, src, expe, w1v, w2v, wvt)


def _final_linear(n0, n1, z0, z1, wl):
    def body(n0_r, n1_r, z0_r, z1_r, wl_r, out_r):
        z = z0_r[...] + z1_r[...]
        w = jnp.where(z == 0.0, 1.0, lax.rsqrt(z))
        ns = n0_r[...] + n1_r[...]
        CH = C // H
        pieces = [ns[:, h * CH:(h + 1) * CH] * w[:, h:h + 1] for h in range(H)]
        scaled = jnp.concatenate(pieces, axis=1)
        out_r[...] = jnp.dot(scaled, wl_r[...],
                             preferred_element_type=jnp.float32)

    return pl.pallas_call(
        body,
        out_shape=jax.ShapeDtypeStruct((N, C), jnp.float32),
    )(n0, n1, z0, z1, wl)


def kernel(edge_src, edge_dst, edge_scalar_attr, edge_weight_cutoff, edge_attr,
           node_feat, W1k, W2k, W1v, W2v, wk, wv, wlogit, wlin):
    edge_src = edge_src.astype(jnp.int32)
    edge_dst = edge_dst.astype(jnp.int32)

    w1k = W1k / np.sqrt(NB)
    w2k = W2k / np.sqrt(NEU)
    w1v = W1v / np.sqrt(NB)
    w2v = W2v / np.sqrt(NEU)
    wkt = (jnp.transpose(wk, (2, 0, 1)).reshape(A * NEU, C)
           / (np.sqrt(NEU) * np.sqrt(A))).astype(jnp.bfloat16)
    wvt = (jnp.transpose(wv, (2, 0, 1)).reshape(A * NEU, C)
           / (np.sqrt(NEU) * np.sqrt(A))).astype(jnp.bfloat16)
    wlt = (jnp.transpose(wlogit, (1, 2, 0)).reshape(C, H * C)
           / C).astype(jnp.bfloat16)
    wl = wlin / np.sqrt(C)

    cut2 = edge_weight_cutoff.reshape(E, 1)

    both = _sc_gather(node_feat, jnp.concatenate([edge_src, edge_dst]),
                      chunk=200)

    expe = _edge_logits(edge_scalar_attr, edge_attr, cut2, both, both,
                        w1k, w2k, wkt, wlt)

    vout = _edge_values(edge_scalar_attr, edge_attr, both, expe,
                        w1v, w2v, wvt)

    zC = jnp.zeros((NP, C), jnp.float32)
    z16 = jnp.zeros((NP, 16), jnp.float32)
    npart, zpart = _sc_scatter_add2(vout, expe, edge_dst, zC, z16, chunk=40)

    return _final_linear(npart[0, :N], npart[1, :N],
                         zpart[0, :N], zpart[1, :N], wl)


# speedup vs baseline: 1.1297x; 1.0090x over previous
import functools

import jax
import jax.numpy as jnp
import numpy as np
from jax import lax
from jax.experimental import pallas as pl
from jax.experimental.pallas import tpu as pltpu
from jax.experimental.pallas import tpu_sc as plsc

N = 10000
E = 160000
C = 128
A = 4
NB = 16
H = 4
NEU = 64

NP = 10112
NW = 32

def _sc_mesh():
    return plsc.VectorSubcoreMesh(core_axis_name="c", subcore_axis_name="s")


def _sc_gather(table, idx, chunk):
    B = idx.shape[0]
    V, D = table.shape
    bpw = B // NW
    nch = bpw // chunk

    @functools.partial(
        pl.kernel, mesh=_sc_mesh(),
        out_type=jax.ShapeDtypeStruct((B, D), table.dtype),
        scratch_types=[
            pltpu.VMEM((chunk,), jnp.int32),
            pltpu.VMEM((chunk,), jnp.int32),
            pltpu.VMEM((2, chunk, D), table.dtype),
            pltpu.SemaphoreType.DMA,
            pltpu.SemaphoreType.DMA,
            pltpu.SemaphoreType.DMA,
            pltpu.SemaphoreType.DMA,
            pltpu.SemaphoreType.DMA,
        ],
    )
    def k(table_hbm, idx_hbm, out_hbm, idx_v0, idx_v1, rows_v,
          sem_i0, sem_i1, sem_g, sem_w0, sem_w1):
        wid = lax.axis_index("s") * 2 + lax.axis_index("c")
        base = wid * bpw
        idx_v = (idx_v0, idx_v1)
        sem_i = (sem_i0, sem_i1)
        sem_w = (sem_w0, sem_w1)

        for b in range(2):
            pltpu.async_copy(idx_hbm.at[pl.ds(base + b * chunk, chunk)],
                             idx_v[b], sem_i[b])

        def step(g, b, prefetch):
            off = base + g * chunk
            pltpu.make_async_copy(idx_hbm.at[pl.ds(off, chunk)],
                                  idx_v[b], sem_i[b]).wait()
            @pl.when(g >= 2)
            def _():
                pltpu.make_async_copy(rows_v.at[b],
                                      out_hbm.at[pl.ds(base, chunk)],
                                      sem_w[b]).wait()
            pltpu.async_copy(table_hbm.at[idx_v[b]], rows_v.at[b],
                             sem_g).wait()
            if prefetch:
                @pl.when(g + 2 < nch)
                def _():
                    pltpu.async_copy(
                        idx_hbm.at[pl.ds(off + 2 * chunk, chunk)],
                        idx_v[b], sem_i[b])
            pltpu.async_copy(rows_v.at[b], out_hbm.at[pl.ds(off, chunk)],
                             sem_w[b])

        @pl.loop(0, nch // 2)
        def _(go):
            for b in range(2):
                step(go * 2 + b, b, True)

        if nch % 2 == 1:
            step(nch - 1, (nch - 1) % 2, False)

        for b in range(2):
            pltpu.make_async_copy(rows_v.at[b],
                                  out_hbm.at[pl.ds(base, chunk)],
                                  sem_w[b]).wait()

    return k(table, idx)


def _sc_scatter_add2(vout, expe, idx, zeros_c, zeros_h, chunk):
    B = idx.shape[0]
    bpw = B // NW
    nch = bpw // chunk
    assert nch % 2 == 1
    rows_pt = NP // 16

    @functools.partial(
        pl.kernel, mesh=_sc_mesh(),
        compiler_params=pltpu.CompilerParams(use_tc_tiling_on_sc=False),
        out_type=(jax.ShapeDtypeStruct((2 * NP, C), jnp.float32),
                  jax.ShapeDtypeStruct((2 * NP, 16), jnp.float32)),
        scratch_types=[
            pltpu.VMEM((chunk,), jnp.int32),
            pltpu.VMEM((chunk,), jnp.int32),
            pltpu.VMEM((2, chunk, C), jnp.float32),
            pltpu.VMEM((2, chunk, 16), jnp.float32),
            pltpu.VMEM_SHARED((NP, C), jnp.float32),
            pltpu.VMEM_SHARED((NP, 16), jnp.float32),
            pltpu.SemaphoreType.DMA,
            pltpu.SemaphoreType.DMA,
        ],
    )
    def k(vout_hbm, expe_hbm, idx_hbm, zc_hbm, zh_hbm, outc_hbm, outh_hbm,
          idx_v0, idx_v1, vc_v, vh_v, acc_c, acc_h, sem_l0, sem_l1):
        idx_v = (idx_v0, idx_v1)
        cid = lax.axis_index("c")
        sid = lax.axis_index("s")
        wid = sid * 2 + cid
        r0 = sid * rows_pt
        base = wid * bpw
        sem_l = (sem_l0, sem_l1)

        def issue_loads(g, b):
            off = base + g * chunk
            pltpu.async_copy(idx_hbm.at[pl.ds(off, chunk)], idx_v[b],
                             sem_l[b])
            pltpu.async_copy(vout_hbm.at[pl.ds(off, chunk)], vc_v.at[b],
                             sem_l[b])
            pltpu.async_copy(expe_hbm.at[pl.ds(off, chunk)], vh_v.at[b],
                             sem_l[b])

        def wait_loads(g, b):
            off = base + g * chunk
            pltpu.make_async_copy(idx_hbm.at[pl.ds(off, chunk)], idx_v[b],
                                  sem_l[b]).wait()
            pltpu.make_async_copy(vout_hbm.at[pl.ds(off, chunk)], vc_v.at[b],
                                  sem_l[b]).wait()
            pltpu.make_async_copy(expe_hbm.at[pl.ds(off, chunk)], vh_v.at[b],
                                  sem_l[b]).wait()

        def do_adds(b):
            pltpu.sync_copy(vc_v.at[b], acc_c.at[idx_v[b]], add=True)
            pltpu.sync_copy(vh_v.at[b], acc_h.at[idx_v[b]], add=True)

        pltpu.sync_copy(zc_hbm.at[pl.ds(r0, rows_pt)],
                        acc_c.at[pl.ds(r0, rows_pt)])
        pltpu.sync_copy(zh_hbm.at[pl.ds(r0, rows_pt)],
                        acc_h.at[pl.ds(r0, rows_pt)])
        plsc.subcore_barrier()

        issue_loads(0, 0)

        @pl.loop(0, (nch - 1) // 2)
        def _(go):
            for b in range(2):
                g = go * 2 + b
                issue_loads_g1 = g + 1
                pltpu.async_copy(
                    idx_hbm.at[pl.ds(base + issue_loads_g1 * chunk, chunk)],
                    idx_v[1 - b], sem_l[1 - b])
                pltpu.async_copy(
                    vout_hbm.at[pl.ds(base + issue_loads_g1 * chunk, chunk)],
                    vc_v.at[1 - b], sem_l[1 - b])
                pltpu.async_copy(
                    expe_hbm.at[pl.ds(base + issue_loads_g1 * chunk, chunk)],
                    vh_v.at[1 - b], sem_l[1 - b])
                wait_loads(g, b)
                do_adds(b)

        wait_loads(nch - 1, (nch - 1) % 2)
        do_adds((nch - 1) % 2)

        plsc.subcore_barrier()
        ro = cid * NP + r0
        pltpu.sync_copy(acc_c.at[pl.ds(r0, rows_pt)],
                        outc_hbm.at[pl.ds(ro, rows_pt)])
        pltpu.sync_copy(acc_h.at[pl.ds(r0, rows_pt)],
                        outh_hbm.at[pl.ds(ro, rows_pt)])

    outc, outh = k(vout, expe, idx, zeros_c, zeros_h)
    return outc.reshape(2, NP, C), outh.reshape(2, NP, 16)


BE = 2000


def _edge_srcside(esa, ea, src, w1k, w2k, w1v, w2v, wkt, wvt):
    def body(esa_r, ea_r, src_r, w1k_r, w2k_r, w1v_r, w2v_r, wkt_r, wvt_r,
             ek_r, ev_r):
        esav = esa_r[...]
        eav = ea_r[...].astype(jnp.bfloat16)
        srcv = src_r[...]

        def tp(w1, w2, wt):
            h = jnp.dot(esav, w1, preferred_element_type=jnp.float32)
            h = jnp.dot(jax.nn.relu(h), w2, preferred_element_type=jnp.float32)
            h = jax.nn.relu(h).astype(jnp.bfloat16)
            h2 = jnp.concatenate([h * eav[:, v:v + 1] for v in range(A)],
                                 axis=1)
            return jnp.dot(h2, wt, preferred_element_type=jnp.float32)

        ek_r[...] = (srcv * tp(w1k_r[...], w2k_r[...], wkt_r[...])
                     ).astype(jnp.bfloat16)
        ev_r[...] = srcv * tp(w1v_r[...], w2v_r[...], wvt_r[...])

    return pl.pallas_call(
        body,
        grid=(E // BE,),
        in_specs=[
            pl.BlockSpec((BE, NB), lambda i: (i, 0)),
            pl.BlockSpec((BE, A), lambda i: (i, 0)),
            pl.BlockSpec((BE, C), lambda i: (i, 0)),
            pl.BlockSpec((NB, NEU), lambda i: (0, 0)),
            pl.BlockSpec((NEU, NEU), lambda i: (0, 0)),
            pl.BlockSpec((NB, NEU), lambda i: (0, 0)),
            pl.BlockSpec((NEU, NEU), lambda i: (0, 0)),
            pl.BlockSpec((A * NEU, C), lambda i: (0, 0)),
            pl.BlockSpec((A * NEU, C), lambda i: (0, 0)),
        ],
        out_specs=(pl.BlockSpec((BE, C), lambda i: (i, 0)),
                   pl.BlockSpec((BE, C), lambda i: (i, 0))),
        out_shape=(jax.ShapeDtypeStruct((E, C), jnp.bfloat16),
                   jax.ShapeDtypeStruct((E, C), jnp.float32)),
    )(esa, ea, src, w1k, w2k, w1v, w2v, wkt, wvt)


def _edge_logits(ek, dst, cut2, wlt):
    def body(ek_r, dst_r, cut_r, wlt_r, out_r):
        dstv = dst_r[...]
        tt = jnp.dot(ek_r[...], wlt_r[...], preferred_element_type=jnp.float32)
        cols = [jnp.sum(dstv * tt[:, h * C:(h + 1) * C], axis=1,
                        keepdims=True) for h in range(H)]
        logit = jnp.concatenate(cols, axis=1)
        e4 = cut_r[...] * jnp.exp(logit)
        pad = jnp.zeros((e4.shape[0], 16 - H), e4.dtype)
        out_r[...] = jnp.concatenate([e4, pad], axis=1)

    return pl.pallas_call(
        body,
        grid=(E // BE,),
        in_specs=[
            pl.BlockSpec((BE, C), lambda i: (i, 0)),
            pl.BlockSpec((BE, C), lambda i: (i, 0)),
            pl.BlockSpec((BE, 1), lambda i: (i, 0)),
            pl.BlockSpec((C, H * C), lambda i: (0, 0)),
        ],
        out_specs=pl.BlockSpec((BE, 16), lambda i: (i, 0)),
        out_shape=jax.ShapeDtypeStruct((E, 16), jnp.float32),
    )(ek, dst, cut2, wlt)


def _edge_weight(ev, expe):
    def body(ev_r, exp_r, out_r):
        evv = ev_r[...]
        w16 = jnp.sqrt(exp_r[...])
        CH = C // H
        pieces = [evv[:, h * CH:(h + 1) * CH] * w16[:, h:h + 1]
                  for h in range(H)]
        out_r[...] = jnp.concatenate(pieces, axis=1)

    return pl.pallas_call(
        body,
        grid=(E // BE,),
        in_specs=[
            pl.BlockSpec((BE, C), lambda i: (i, 0)),
            pl.BlockSpec((BE, 16), lambda i: (i, 0)),
        ],
        out_specs=pl.BlockSpec((BE, C), lambda i: (i, 0)),
        out_shape=jax.ShapeDtypeStruct((E, C), jnp.float32),
    )(ev, expe)


def _final_linear(n0, n1, z0, z1, wl):
    def body(n0_r, n1_r, z0_r, z1_r, wl_r, out_r):
        z = z0_r[...] + z1_r[...]
        w = jnp.where(z == 0.0, 1.0, lax.rsqrt(z))
        ns = n0_r[...] + n1_r[...]
        CH = C // H
        pieces = [ns[:, h * CH:(h + 1) * CH] * w[:, h:h + 1] for h in range(H)]
        scaled = jnp.concatenate(pieces, axis=1)
        out_r[...] = jnp.dot(scaled, wl_r[...],
                             preferred_element_type=jnp.float32)

    return pl.pallas_call(
        body,
        out_shape=jax.ShapeDtypeStruct((N, C), jnp.float32),
    )(n0, n1, z0, z1, wl)


def kernel(edge_src, edge_dst, edge_scalar_attr, edge_weight_cutoff, edge_attr,
           node_feat, W1k, W2k, W1v, W2v, wk, wv, wlogit, wlin):
    edge_src = edge_src.astype(jnp.int32)
    edge_dst = edge_dst.astype(jnp.int32)

    w1k = W1k / np.sqrt(NB)
    w2k = W2k / np.sqrt(NEU)
    w1v = W1v / np.sqrt(NB)
    w2v = W2v / np.sqrt(NEU)
    wkt = (jnp.transpose(wk, (2, 0, 1)).reshape(A * NEU, C)
           / (np.sqrt(NEU) * np.sqrt(A))).astype(jnp.bfloat16)
    wvt = (jnp.transpose(wv, (2, 0, 1)).reshape(A * NEU, C)
           / (np.sqrt(NEU) * np.sqrt(A))).astype(jnp.bfloat16)
    wlt = (jnp.transpose(wlogit, (1, 2, 0)).reshape(C, H * C)
           / C).astype(jnp.bfloat16)
    wl = wlin / np.sqrt(C)

    cut2 = edge_weight_cutoff.reshape(E, 1)

    src_feat = _sc_gather(node_feat, edge_src, chunk=200)
    dst_feat = _sc_gather(node_feat, edge_dst, chunk=200)

    ek, ev = _edge_srcside(edge_scalar_attr, edge_attr, src_feat,
                           w1k, w2k, w1v, w2v, wkt, wvt)
    expe = _edge_logits(ek, dst_feat, cut2, wlt)
    vout = _edge_weight(ev, expe)

    zC = jnp.zeros((NP, C), jnp.float32)
    z16 = jnp.zeros((NP, 16), jnp.float32)
    npart, zpart = _sc_scatter_add2(vout, expe, edge_dst, zC, z16, chunk=40)

    return _final_linear(npart[0, :N], npart[1, :N],
                         zpart[0, :N], zpart[1, :N], wl)


# speedup vs baseline: 1.2322x; 1.0907x over previous
import functools

import jax
import jax.numpy as jnp
import numpy as np
from jax import lax
from jax.experimental import pallas as pl
from jax.experimental.pallas import tpu as pltpu
from jax.experimental.pallas import tpu_sc as plsc

N = 10000
E = 160000
C = 128
A = 4
NB = 16
H = 4
NEU = 64

NP = 10112
NW = 32

def _sc_mesh():
    return plsc.VectorSubcoreMesh(core_axis_name="c", subcore_axis_name="s")


def _sc_gather(table, idx, chunk):
    B = idx.shape[0]
    V, D = table.shape
    bpw = B // NW
    nch = bpw // chunk

    @functools.partial(
        pl.kernel, mesh=_sc_mesh(),
        out_type=jax.ShapeDtypeStruct((B, D), table.dtype),
        scratch_types=[
            pltpu.VMEM((chunk,), jnp.int32),
            pltpu.VMEM((chunk,), jnp.int32),
            pltpu.VMEM((2, chunk, D), table.dtype),
            pltpu.SemaphoreType.DMA,
            pltpu.SemaphoreType.DMA,
            pltpu.SemaphoreType.DMA,
            pltpu.SemaphoreType.DMA,
            pltpu.SemaphoreType.DMA,
        ],
    )
    def k(table_hbm, idx_hbm, out_hbm, idx_v0, idx_v1, rows_v,
          sem_i0, sem_i1, sem_g, sem_w0, sem_w1):
        wid = lax.axis_index("s") * 2 + lax.axis_index("c")
        base = wid * bpw
        idx_v = (idx_v0, idx_v1)
        sem_i = (sem_i0, sem_i1)
        sem_w = (sem_w0, sem_w1)

        for b in range(2):
            pltpu.async_copy(idx_hbm.at[pl.ds(base + b * chunk, chunk)],
                             idx_v[b], sem_i[b])

        def step(g, b, prefetch):
            off = base + g * chunk
            pltpu.make_async_copy(idx_hbm.at[pl.ds(off, chunk)],
                                  idx_v[b], sem_i[b]).wait()
            @pl.when(g >= 2)
            def _():
                pltpu.make_async_copy(rows_v.at[b],
                                      out_hbm.at[pl.ds(base, chunk)],
                                      sem_w[b]).wait()
            pltpu.async_copy(table_hbm.at[idx_v[b]], rows_v.at[b],
                             sem_g).wait()
            if prefetch:
                @pl.when(g + 2 < nch)
                def _():
                    pltpu.async_copy(
                        idx_hbm.at[pl.ds(off + 2 * chunk, chunk)],
                        idx_v[b], sem_i[b])
            pltpu.async_copy(rows_v.at[b], out_hbm.at[pl.ds(off, chunk)],
                             sem_w[b])

        @pl.loop(0, nch // 2)
        def _(go):
            for b in range(2):
                step(go * 2 + b, b, True)

        if nch % 2 == 1:
            step(nch - 1, (nch - 1) % 2, False)

        for b in range(2):
            pltpu.make_async_copy(rows_v.at[b],
                                  out_hbm.at[pl.ds(base, chunk)],
                                  sem_w[b]).wait()

    return k(table, idx)


def _sc_scatter_add2(vout, expe, idx, zeros_c, zeros_h, chunk):
    B = idx.shape[0]
    bpw = B // NW
    nch = bpw // chunk
    assert nch % 2 == 1
    rows_pt = NP // 16

    @functools.partial(
        pl.kernel, mesh=_sc_mesh(),
        compiler_params=pltpu.CompilerParams(use_tc_tiling_on_sc=False),
        out_type=(jax.ShapeDtypeStruct((2 * NP, C), jnp.float32),
                  jax.ShapeDtypeStruct((2 * NP, 16), jnp.float32)),
        scratch_types=[
            pltpu.VMEM((chunk,), jnp.int32),
            pltpu.VMEM((chunk,), jnp.int32),
            pltpu.VMEM((2, chunk, C), jnp.float32),
            pltpu.VMEM((2, chunk, 16), jnp.float32),
            pltpu.VMEM_SHARED((NP, C), jnp.float32),
            pltpu.VMEM_SHARED((NP, 16), jnp.float32),
            pltpu.SemaphoreType.DMA,
            pltpu.SemaphoreType.DMA,
        ],
    )
    def k(vout_hbm, expe_hbm, idx_hbm, zc_hbm, zh_hbm, outc_hbm, outh_hbm,
          idx_v0, idx_v1, vc_v, vh_v, acc_c, acc_h, sem_l0, sem_l1):
        idx_v = (idx_v0, idx_v1)
        cid = lax.axis_index("c")
        sid = lax.axis_index("s")
        wid = sid * 2 + cid
        r0 = sid * rows_pt
        base = wid * bpw
        sem_l = (sem_l0, sem_l1)

        def issue_loads(g, b):
            off = base + g * chunk
            pltpu.async_copy(idx_hbm.at[pl.ds(off, chunk)], idx_v[b],
                             sem_l[b])
            pltpu.async_copy(vout_hbm.at[pl.ds(off, chunk)], vc_v.at[b],
                             sem_l[b])
            pltpu.async_copy(expe_hbm.at[pl.ds(off, chunk)], vh_v.at[b],
                             sem_l[b])

        def wait_loads(g, b):
            off = base + g * chunk
            pltpu.make_async_copy(idx_hbm.at[pl.ds(off, chunk)], idx_v[b],
                                  sem_l[b]).wait()
            pltpu.make_async_copy(vout_hbm.at[pl.ds(off, chunk)], vc_v.at[b],
                                  sem_l[b]).wait()
            pltpu.make_async_copy(expe_hbm.at[pl.ds(off, chunk)], vh_v.at[b],
                                  sem_l[b]).wait()

        def do_adds(b):
            pltpu.sync_copy(vc_v.at[b], acc_c.at[idx_v[b]], add=True)
            pltpu.sync_copy(vh_v.at[b], acc_h.at[idx_v[b]], add=True)

        pltpu.sync_copy(zc_hbm.at[pl.ds(r0, rows_pt)],
                        acc_c.at[pl.ds(r0, rows_pt)])
        pltpu.sync_copy(zh_hbm.at[pl.ds(r0, rows_pt)],
                        acc_h.at[pl.ds(r0, rows_pt)])
        plsc.subcore_barrier()

        issue_loads(0, 0)

        @pl.loop(0, (nch - 1) // 2)
        def _(go):
            for b in range(2):
                g = go * 2 + b
                issue_loads_g1 = g + 1
                pltpu.async_copy(
                    idx_hbm.at[pl.ds(base + issue_loads_g1 * chunk, chunk)],
                    idx_v[1 - b], sem_l[1 - b])
                pltpu.async_copy(
                    vout_hbm.at[pl.ds(base + issue_loads_g1 * chunk, chunk)],
                    vc_v.at[1 - b], sem_l[1 - b])
                pltpu.async_copy(
                    expe_hbm.at[pl.ds(base + issue_loads_g1 * chunk, chunk)],
                    vh_v.at[1 - b], sem_l[1 - b])
                wait_loads(g, b)
                do_adds(b)

        wait_loads(nch - 1, (nch - 1) % 2)
        do_adds((nch - 1) % 2)

        plsc.subcore_barrier()
        ro = cid * NP + r0
        pltpu.sync_copy(acc_c.at[pl.ds(r0, rows_pt)],
                        outc_hbm.at[pl.ds(ro, rows_pt)])
        pltpu.sync_copy(acc_h.at[pl.ds(r0, rows_pt)],
                        outh_hbm.at[pl.ds(ro, rows_pt)])

    outc, outh = k(vout, expe, idx, zeros_c, zeros_h)
    return outc.reshape(2, NP, C), outh.reshape(2, NP, 16)


BE = 4000


def _edge_srcside(esa, ea, src, w1k, w2k, w1v, w2v, wkt, wvt):
    def body(esa_r, ea_r, src_r, w1k_r, w2k_r, w1v_r, w2v_r, wkt_r, wvt_r,
             ek_r, ev_r):
        esav = esa_r[...]
        eav = ea_r[...].astype(jnp.bfloat16)
        srcv = src_r[...]

        def tp(w1, w2, wt):
            h = jnp.dot(esav, w1, preferred_element_type=jnp.float32)
            h = jnp.dot(jax.nn.relu(h), w2, preferred_element_type=jnp.float32)
            h = jax.nn.relu(h).astype(jnp.bfloat16)
            h2 = jnp.concatenate([h * eav[:, v:v + 1] for v in range(A)],
                                 axis=1)
            return jnp.dot(h2, wt, preferred_element_type=jnp.float32)

        ek_r[...] = (srcv * tp(w1k_r[...], w2k_r[...], wkt_r[...])
                     ).astype(jnp.bfloat16)
        ev_r[...] = srcv * tp(w1v_r[...], w2v_r[...], wvt_r[...])

    return pl.pallas_call(
        body,
        grid=(E // BE,),
        in_specs=[
            pl.BlockSpec((BE, NB), lambda i: (i, 0)),
            pl.BlockSpec((BE, A), lambda i: (i, 0)),
            pl.BlockSpec((BE, C), lambda i: (i, 0)),
            pl.BlockSpec((NB, NEU), lambda i: (0, 0)),
            pl.BlockSpec((NEU, NEU), lambda i: (0, 0)),
            pl.BlockSpec((NB, NEU), lambda i: (0, 0)),
            pl.BlockSpec((NEU, NEU), lambda i: (0, 0)),
            pl.BlockSpec((A * NEU, C), lambda i: (0, 0)),
            pl.BlockSpec((A * NEU, C), lambda i: (0, 0)),
        ],
        out_specs=(pl.BlockSpec((BE, C), lambda i: (i, 0)),
                   pl.BlockSpec((BE, C), lambda i: (i, 0))),
        out_shape=(jax.ShapeDtypeStruct((E, C), jnp.bfloat16),
                   jax.ShapeDtypeStruct((E, C), jnp.float32)),
    )(esa, ea, src, w1k, w2k, w1v, w2v, wkt, wvt)


def _edge_logits(ek, dst, cut2, wlt):
    def body(ek_r, dst_r, cut_r, wlt_r, out_r):
        dstv = dst_r[...]
        tt = jnp.dot(ek_r[...], wlt_r[...], preferred_element_type=jnp.float32)
        cols = [jnp.sum(dstv * tt[:, h * C:(h + 1) * C], axis=1,
                        keepdims=True) for h in range(H)]
        logit = jnp.concatenate(cols, axis=1)
        e4 = cut_r[...] * jnp.exp(logit)
        pad = jnp.zeros((e4.shape[0], 16 - H), e4.dtype)
        out_r[...] = jnp.concatenate([e4, pad], axis=1)

    return pl.pallas_call(
        body,
        grid=(E // BE,),
        in_specs=[
            pl.BlockSpec((BE, C), lambda i: (i, 0)),
            pl.BlockSpec((BE, C), lambda i: (i, 0)),
            pl.BlockSpec((BE, 1), lambda i: (i, 0)),
            pl.BlockSpec((C, H * C), lambda i: (0, 0)),
        ],
        out_specs=pl.BlockSpec((BE, 16), lambda i: (i, 0)),
        out_shape=jax.ShapeDtypeStruct((E, 16), jnp.float32),
    )(ek, dst, cut2, wlt)


def _edge_weight(ev, expe):
    def body(ev_r, exp_r, out_r):
        evv = ev_r[...]
        w16 = jnp.sqrt(exp_r[...])
        CH = C // H
        pieces = [evv[:, h * CH:(h + 1) * CH] * w16[:, h:h + 1]
                  for h in range(H)]
        out_r[...] = jnp.concatenate(pieces, axis=1)

    return pl.pallas_call(
        body,
        grid=(E // BE,),
        in_specs=[
            pl.BlockSpec((BE, C), lambda i: (i, 0)),
            pl.BlockSpec((BE, 16), lambda i: (i, 0)),
        ],
        out_specs=pl.BlockSpec((BE, C), lambda i: (i, 0)),
        out_shape=jax.ShapeDtypeStruct((E, C), jnp.float32),
    )(ev, expe)


def _final_linear(n0, n1, z0, z1, wl):
    def body(n0_r, n1_r, z0_r, z1_r, wl_r, out_r):
        z = z0_r[...] + z1_r[...]
        w = jnp.where(z == 0.0, 1.0, lax.rsqrt(z))
        ns = n0_r[...] + n1_r[...]
        CH = C // H
        pieces = [ns[:, h * CH:(h + 1) * CH] * w[:, h:h + 1] for h in range(H)]
        scaled = jnp.concatenate(pieces, axis=1)
        out_r[...] = jnp.dot(scaled, wl_r[...],
                             preferred_element_type=jnp.float32)

    return pl.pallas_call(
        body,
        out_shape=jax.ShapeDtypeStruct((N, C), jnp.float32),
    )(n0, n1, z0, z1, wl)


def kernel(edge_src, edge_dst, edge_scalar_attr, edge_weight_cutoff, edge_attr,
           node_feat, W1k, W2k, W1v, W2v, wk, wv, wlogit, wlin):
    edge_src = edge_src.astype(jnp.int32)
    edge_dst = edge_dst.astype(jnp.int32)

    w1k = W1k / np.sqrt(NB)
    w2k = W2k / np.sqrt(NEU)
    w1v = W1v / np.sqrt(NB)
    w2v = W2v / np.sqrt(NEU)
    wkt = (jnp.transpose(wk, (2, 0, 1)).reshape(A * NEU, C)
           / (np.sqrt(NEU) * np.sqrt(A))).astype(jnp.bfloat16)
    wvt = (jnp.transpose(wv, (2, 0, 1)).reshape(A * NEU, C)
           / (np.sqrt(NEU) * np.sqrt(A))).astype(jnp.bfloat16)
    wlt = (jnp.transpose(wlogit, (1, 2, 0)).reshape(C, H * C)
           / C).astype(jnp.bfloat16)
    wl = wlin / np.sqrt(C)

    cut2 = edge_weight_cutoff.reshape(E, 1)

    src_feat = _sc_gather(node_feat, edge_src, chunk=200)
    dst_feat = _sc_gather(node_feat, edge_dst, chunk=200)

    ek, ev = _edge_srcside(edge_scalar_attr, edge_attr, src_feat,
                           w1k, w2k, w1v, w2v, wkt, wvt)
    expe = _edge_logits(ek, dst_feat, cut2, wlt)
    vout = _edge_weight(ev, expe)

    zC = jnp.zeros((NP, C), jnp.float32)
    z16 = jnp.zeros((NP, 16), jnp.float32)
    npart, zpart = _sc_scatter_add2(vout, expe, edge_dst, zC, z16, chunk=40)

    return _final_linear(npart[0, :N], npart[1, :N],
                         zpart[0, :N], zpart[1, :N], wl)


# speedup vs baseline: 1.2696x; 1.0303x over previous
import functools

import jax
import jax.numpy as jnp
import numpy as np
from jax import lax
from jax.experimental import pallas as pl
from jax.experimental.pallas import tpu as pltpu
from jax.experimental.pallas import tpu_sc as plsc

N = 10000
E = 160000
C = 128
A = 4
NB = 16
H = 4
NEU = 64

NP = 10112
NW = 32

def _sc_mesh():
    return plsc.VectorSubcoreMesh(core_axis_name="c", subcore_axis_name="s")


def _sc_gather(table, idx, chunk):
    B = idx.shape[0]
    V, D = table.shape
    bpw = B // NW
    nch = bpw // chunk

    @functools.partial(
        pl.kernel, mesh=_sc_mesh(),
        out_type=jax.ShapeDtypeStruct((B, D), table.dtype),
        scratch_types=[
            pltpu.VMEM((chunk,), jnp.int32),
            pltpu.VMEM((chunk,), jnp.int32),
            pltpu.VMEM((2, chunk, D), table.dtype),
            pltpu.SemaphoreType.DMA,
            pltpu.SemaphoreType.DMA,
            pltpu.SemaphoreType.DMA,
            pltpu.SemaphoreType.DMA,
            pltpu.SemaphoreType.DMA,
        ],
    )
    def k(table_hbm, idx_hbm, out_hbm, idx_v0, idx_v1, rows_v,
          sem_i0, sem_i1, sem_g, sem_w0, sem_w1):
        wid = lax.axis_index("s") * 2 + lax.axis_index("c")
        base = wid * bpw
        idx_v = (idx_v0, idx_v1)
        sem_i = (sem_i0, sem_i1)
        sem_w = (sem_w0, sem_w1)

        for b in range(2):
            pltpu.async_copy(idx_hbm.at[pl.ds(base + b * chunk, chunk)],
                             idx_v[b], sem_i[b])

        def step(g, b, prefetch):
            off = base + g * chunk
            pltpu.make_async_copy(idx_hbm.at[pl.ds(off, chunk)],
                                  idx_v[b], sem_i[b]).wait()
            @pl.when(g >= 2)
            def _():
                pltpu.make_async_copy(rows_v.at[b],
                                      out_hbm.at[pl.ds(base, chunk)],
                                      sem_w[b]).wait()
            pltpu.async_copy(table_hbm.at[idx_v[b]], rows_v.at[b],
                             sem_g).wait()
            if prefetch:
                @pl.when(g + 2 < nch)
                def _():
                    pltpu.async_copy(
                        idx_hbm.at[pl.ds(off + 2 * chunk, chunk)],
                        idx_v[b], sem_i[b])
            pltpu.async_copy(rows_v.at[b], out_hbm.at[pl.ds(off, chunk)],
                             sem_w[b])

        @pl.loop(0, nch // 2)
        def _(go):
            for b in range(2):
                step(go * 2 + b, b, True)

        if nch % 2 == 1:
            step(nch - 1, (nch - 1) % 2, False)

        for b in range(2):
            pltpu.make_async_copy(rows_v.at[b],
                                  out_hbm.at[pl.ds(base, chunk)],
                                  sem_w[b]).wait()

    return k(table, idx)


def _sc_scatter_add2(vout, expe, idx, zeros_c, zeros_h, chunk):
    B = idx.shape[0]
    bpw = B // NW
    nch = bpw // chunk
    assert nch % 2 == 1
    rows_pt = NP // 16

    @functools.partial(
        pl.kernel, mesh=_sc_mesh(),
        compiler_params=pltpu.CompilerParams(use_tc_tiling_on_sc=False),
        out_type=(jax.ShapeDtypeStruct((2 * NP, C), jnp.float32),
                  jax.ShapeDtypeStruct((2 * NP, 16), jnp.float32)),
        scratch_types=[
            pltpu.VMEM((chunk,), jnp.int32),
            pltpu.VMEM((chunk,), jnp.int32),
            pltpu.VMEM((2, chunk, C), jnp.float32),
            pltpu.VMEM((2, chunk, 16), jnp.float32),
            pltpu.VMEM_SHARED((NP, C), jnp.float32),
            pltpu.VMEM_SHARED((NP, 16), jnp.float32),
            pltpu.SemaphoreType.DMA,
            pltpu.SemaphoreType.DMA,
        ],
    )
    def k(vout_hbm, expe_hbm, idx_hbm, zc_hbm, zh_hbm, outc_hbm, outh_hbm,
          idx_v0, idx_v1, vc_v, vh_v, acc_c, acc_h, sem_l0, sem_l1):
        idx_v = (idx_v0, idx_v1)
        cid = lax.axis_index("c")
        sid = lax.axis_index("s")
        wid = sid * 2 + cid
        r0 = sid * rows_pt
        base = wid * bpw
        sem_l = (sem_l0, sem_l1)

        def issue_loads(g, b):
            off = base + g * chunk
            pltpu.async_copy(idx_hbm.at[pl.ds(off, chunk)], idx_v[b],
                             sem_l[b])
            pltpu.async_copy(vout_hbm.at[pl.ds(off, chunk)], vc_v.at[b],
                             sem_l[b])
            pltpu.async_copy(expe_hbm.at[pl.ds(off, chunk)], vh_v.at[b],
                             sem_l[b])

        def wait_loads(g, b):
            off = base + g * chunk
            pltpu.make_async_copy(idx_hbm.at[pl.ds(off, chunk)], idx_v[b],
                                  sem_l[b]).wait()
            pltpu.make_async_copy(vout_hbm.at[pl.ds(off, chunk)], vc_v.at[b],
                                  sem_l[b]).wait()
            pltpu.make_async_copy(expe_hbm.at[pl.ds(off, chunk)], vh_v.at[b],
                                  sem_l[b]).wait()

        def do_adds(b):
            pltpu.sync_copy(vc_v.at[b], acc_c.at[idx_v[b]], add=True)
            pltpu.sync_copy(vh_v.at[b], acc_h.at[idx_v[b]], add=True)

        pltpu.sync_copy(zc_hbm.at[pl.ds(r0, rows_pt)],
                        acc_c.at[pl.ds(r0, rows_pt)])
        pltpu.sync_copy(zh_hbm.at[pl.ds(r0, rows_pt)],
                        acc_h.at[pl.ds(r0, rows_pt)])
        plsc.subcore_barrier()

        issue_loads(0, 0)

        @pl.loop(0, (nch - 1) // 2)
        def _(go):
            for b in range(2):
                g = go * 2 + b
                issue_loads_g1 = g + 1
                pltpu.async_copy(
                    idx_hbm.at[pl.ds(base + issue_loads_g1 * chunk, chunk)],
                    idx_v[1 - b], sem_l[1 - b])
                pltpu.async_copy(
                    vout_hbm.at[pl.ds(base + issue_loads_g1 * chunk, chunk)],
                    vc_v.at[1 - b], sem_l[1 - b])
                pltpu.async_copy(
                    expe_hbm.at[pl.ds(base + issue_loads_g1 * chunk, chunk)],
                    vh_v.at[1 - b], sem_l[1 - b])
                wait_loads(g, b)
                do_adds(b)

        wait_loads(nch - 1, (nch - 1) % 2)
        do_adds((nch - 1) % 2)

        plsc.subcore_barrier()
        ro = cid * NP + r0
        pltpu.sync_copy(acc_c.at[pl.ds(r0, rows_pt)],
                        outc_hbm.at[pl.ds(ro, rows_pt)])
        pltpu.sync_copy(acc_h.at[pl.ds(r0, rows_pt)],
                        outh_hbm.at[pl.ds(ro, rows_pt)])

    outc, outh = k(vout, expe, idx, zeros_c, zeros_h)
    return outc.reshape(2, NP, C), outh.reshape(2, NP, 16)


BE = 8000


def _edge_srcside(esa, ea, src, w1k, w2k, w1v, w2v, wkt, wvt):
    def body(esa_r, ea_r, src_r, w1k_r, w2k_r, w1v_r, w2v_r, wkt_r, wvt_r,
             ek_r, ev_r):
        esav = esa_r[...]
        eav = ea_r[...].astype(jnp.bfloat16)
        srcv = src_r[...]

        def tp(w1, w2, wt):
            h = jnp.dot(esav, w1, preferred_element_type=jnp.float32)
            h = jnp.dot(jax.nn.relu(h), w2, preferred_element_type=jnp.float32)
            h = jax.nn.relu(h).astype(jnp.bfloat16)
            h2 = jnp.concatenate([h * eav[:, v:v + 1] for v in range(A)],
                                 axis=1)
            return jnp.dot(h2, wt, preferred_element_type=jnp.float32)

        ek_r[...] = (srcv * tp(w1k_r[...], w2k_r[...], wkt_r[...])
                     ).astype(jnp.bfloat16)
        ev_r[...] = srcv * tp(w1v_r[...], w2v_r[...], wvt_r[...])

    return pl.pallas_call(
        body,
        grid=(E // BE,),
        in_specs=[
            pl.BlockSpec((BE, NB), lambda i: (i, 0)),
            pl.BlockSpec((BE, A), lambda i: (i, 0)),
            pl.BlockSpec((BE, C), lambda i: (i, 0)),
            pl.BlockSpec((NB, NEU), lambda i: (0, 0)),
            pl.BlockSpec((NEU, NEU), lambda i: (0, 0)),
            pl.BlockSpec((NB, NEU), lambda i: (0, 0)),
            pl.BlockSpec((NEU, NEU), lambda i: (0, 0)),
            pl.BlockSpec((A * NEU, C), lambda i: (0, 0)),
            pl.BlockSpec((A * NEU, C), lambda i: (0, 0)),
        ],
        out_specs=(pl.BlockSpec((BE, C), lambda i: (i, 0)),
                   pl.BlockSpec((BE, C), lambda i: (i, 0))),
        out_shape=(jax.ShapeDtypeStruct((E, C), jnp.bfloat16),
                   jax.ShapeDtypeStruct((E, C), jnp.float32)),
    )(esa, ea, src, w1k, w2k, w1v, w2v, wkt, wvt)


def _edge_logits(ek, dst, cut2, wlt):
    def body(ek_r, dst_r, cut_r, wlt_r, out_r):
        dstv = dst_r[...]
        tt = jnp.dot(ek_r[...], wlt_r[...], preferred_element_type=jnp.float32)
        cols = [jnp.sum(dstv * tt[:, h * C:(h + 1) * C], axis=1,
                        keepdims=True) for h in range(H)]
        logit = jnp.concatenate(cols, axis=1)
        e4 = cut_r[...] * jnp.exp(logit)
        pad = jnp.zeros((e4.shape[0], 16 - H), e4.dtype)
        out_r[...] = jnp.concatenate([e4, pad], axis=1)

    return pl.pallas_call(
        body,
        grid=(E // BE,),
        in_specs=[
            pl.BlockSpec((BE, C), lambda i: (i, 0)),
            pl.BlockSpec((BE, C), lambda i: (i, 0)),
            pl.BlockSpec((BE, 1), lambda i: (i, 0)),
            pl.BlockSpec((C, H * C), lambda i: (0, 0)),
        ],
        out_specs=pl.BlockSpec((BE, 16), lambda i: (i, 0)),
        out_shape=jax.ShapeDtypeStruct((E, 16), jnp.float32),
    )(ek, dst, cut2, wlt)


def _edge_weight(ev, expe):
    def body(ev_r, exp_r, out_r):
        evv = ev_r[...]
        w16 = jnp.sqrt(exp_r[...])
        CH = C // H
        pieces = [evv[:, h * CH:(h + 1) * CH] * w16[:, h:h + 1]
                  for h in range(H)]
        out_r[...] = jnp.concatenate(pieces, axis=1)

    return pl.pallas_call(
        body,
        grid=(E // BE,),
        in_specs=[
            pl.BlockSpec((BE, C), lambda i: (i, 0)),
            pl.BlockSpec((BE, 16), lambda i: (i, 0)),
        ],
        out_specs=pl.BlockSpec((BE, C), lambda i: (i, 0)),
        out_shape=jax.ShapeDtypeStruct((E, C), jnp.float32),
    )(ev, expe)


def _final_linear(n0, n1, z0, z1, wl):
    def body(n0_r, n1_r, z0_r, z1_r, wl_r, out_r):
        z = z0_r[...] + z1_r[...]
        w = jnp.where(z == 0.0, 1.0, lax.rsqrt(z))
        ns = n0_r[...] + n1_r[...]
        CH = C // H
        pieces = [ns[:, h * CH:(h + 1) * CH] * w[:, h:h + 1] for h in range(H)]
        scaled = jnp.concatenate(pieces, axis=1)
        out_r[...] = jnp.dot(scaled, wl_r[...],
                             preferred_element_type=jnp.float32)

    return pl.pallas_call(
        body,
        out_shape=jax.ShapeDtypeStruct((N, C), jnp.float32),
    )(n0, n1, z0, z1, wl)


def kernel(edge_src, edge_dst, edge_scalar_attr, edge_weight_cutoff, edge_attr,
           node_feat, W1k, W2k, W1v, W2v, wk, wv, wlogit, wlin):
    edge_src = edge_src.astype(jnp.int32)
    edge_dst = edge_dst.astype(jnp.int32)

    w1k = W1k / np.sqrt(NB)
    w2k = W2k / np.sqrt(NEU)
    w1v = W1v / np.sqrt(NB)
    w2v = W2v / np.sqrt(NEU)
    wkt = (jnp.transpose(wk, (2, 0, 1)).reshape(A * NEU, C)
           / (np.sqrt(NEU) * np.sqrt(A))).astype(jnp.bfloat16)
    wvt = (jnp.transpose(wv, (2, 0, 1)).reshape(A * NEU, C)
           / (np.sqrt(NEU) * np.sqrt(A))).astype(jnp.bfloat16)
    wlt = (jnp.transpose(wlogit, (1, 2, 0)).reshape(C, H * C)
           / C).astype(jnp.bfloat16)
    wl = wlin / np.sqrt(C)

    cut2 = edge_weight_cutoff.reshape(E, 1)

    src_feat = _sc_gather(node_feat, edge_src, chunk=200)
    dst_feat = _sc_gather(node_feat, edge_dst, chunk=200)

    ek, ev = _edge_srcside(edge_scalar_attr, edge_attr, src_feat,
                           w1k, w2k, w1v, w2v, wkt, wvt)
    expe = _edge_logits(ek, dst_feat, cut2, wlt)
    vout = _edge_weight(ev, expe)

    zC = jnp.zeros((NP, C), jnp.float32)
    z16 = jnp.zeros((NP, 16), jnp.float32)
    npart, zpart = _sc_scatter_add2(vout, expe, edge_dst, zC, z16, chunk=40)

    return _final_linear(npart[0, :N], npart[1, :N],
                         zpart[0, :N], zpart[1, :N], wl)
